# baseline (speedup 1.0000x reference)
import jax
import jax.numpy as jnp
from jax.experimental import pallas as pl
from jax.experimental.pallas import tpu as pltpu

_N = 2048
_D = 64


def _gcn_body(x_ref, adj_ref, w_ref, b_ref, v_ref, c_ref, out_ref):
    adj = adj_ref[0]
    H = _N // 2

    def gconv(h, w, b, dout):
        agg = jnp.dot(adj, h, preferred_element_type=jnp.float32)
        ut = jnp.concatenate([h[:H], agg[:H], h[H:], agg[H:]], axis=1)
        z = jnp.dot(ut, w, preferred_element_type=jnp.float32)
        return jnp.concatenate([z[:, :dout], z[:, dout:]], axis=0) + b

    h = gconv(x_ref[0], w_ref[0], b_ref[0], _D)
    for r in range(6):
        i, j = 1 + 2 * r, 2 + 2 * r
        o1 = jnp.maximum(gconv(h, w_ref[i], b_ref[i], _D), 0.0)
        h = jnp.maximum(gconv(o1, w_ref[j], b_ref[j], _D) + h, 0.0)
    out_ref[0] = gconv(h, v_ref[0], c_ref[0], 2)


def kernel(x, adj, params):
    B = x.shape[0]
    L = params["layers"]

    def blockdiag2(w12):
        k, d = w12.shape
        z = jnp.zeros((k, d), w12.dtype)
        return jnp.concatenate([
            jnp.concatenate([w12, z], axis=1),
            jnp.concatenate([z, w12], axis=1)], axis=0)

    w = jnp.stack([blockdiag2(jnp.concatenate([l["W1"], l["W2"]], axis=0))
                   for l in L[:13]])
    b = jnp.stack([l["b1"] + l["b2"] for l in L[:13]])[:, None, :]
    v13 = jnp.concatenate([L[13]["W1"], L[13]["W2"]], axis=0)
    b13 = L[13]["b1"] + L[13]["b2"]
    v = blockdiag2(v13 @ params["fcW"])[None]
    c = (b13 @ params["fcW"] + params["fcb"])[None, None, :]

    full = lambda s: pl.BlockSpec(s, lambda g: (0,) * len(s))
    grid_spec = pl.GridSpec(
        grid=(B,),
        in_specs=[
            pl.BlockSpec((1, _N, _D), lambda g: (g, 0, 0)),
            pl.BlockSpec((1, _N, _N), lambda g: (g, 0, 0)),
            full((13, 4 * _D, 2 * _D)), full((13, 1, _D)),
            full((1, 4 * _D, 4)), full((1, 1, 2)),
        ],
        out_specs=pl.BlockSpec((1, _N, 2), lambda g: (g, 0, 0)),
    )
    return pl.pallas_call(
        _gcn_body,
        grid_spec=grid_spec,
        out_shape=jax.ShapeDtypeStruct((B, _N, 2), jnp.float32),
        compiler_params=pltpu.CompilerParams(
            dimension_semantics=("arbitrary",),
            vmem_limit_bytes=100 * 1024 * 1024,
        ),
    )(x, adj, w, b, v, c)


# speedup vs baseline: 1.0295x; 1.0295x over previous
import jax
import jax.numpy as jnp
from jax.experimental import pallas as pl
from jax.experimental.pallas import tpu as pltpu

_N = 2048
_D = 64


def _gcn_body(x_ref, adj_ref, w_ref, b_ref, v_ref, c_ref, out_ref):
    adj = adj_ref[0]

    def gconv(h, w, b):
        agg = jnp.dot(adj, h, preferred_element_type=jnp.float32)
        u = jnp.concatenate([h, agg], axis=1)
        return jnp.dot(u, w, preferred_element_type=jnp.float32) + b

    h = gconv(x_ref[0], w_ref[0], b_ref[0])
    for r in range(6):
        i, j = 1 + 2 * r, 2 + 2 * r
        o1 = jnp.maximum(gconv(h, w_ref[i], b_ref[i]), 0.0)
        h = jnp.maximum(gconv(o1, w_ref[j], b_ref[j]) + h, 0.0)
    out_ref[0] = gconv(h, v_ref[0], c_ref[0])


def kernel(x, adj, params):
    B = x.shape[0]
    L = params["layers"]
    w = jnp.stack([jnp.concatenate([l["W1"], l["W2"]], axis=0)
                   for l in L[:13]])
    b = jnp.stack([l["b1"] + l["b2"] for l in L[:13]])[:, None, :]
    v13 = jnp.concatenate([L[13]["W1"], L[13]["W2"]], axis=0)
    b13 = L[13]["b1"] + L[13]["b2"]
    v = (v13 @ params["fcW"])[None]
    c = (b13 @ params["fcW"] + params["fcb"])[None, None, :]

    full = lambda s: pl.BlockSpec(s, lambda g: (0,) * len(s))
    grid_spec = pl.GridSpec(
        grid=(B,),
        in_specs=[
            pl.BlockSpec((1, _N, _D), lambda g: (g, 0, 0)),
            pl.BlockSpec((1, _N, _N), lambda g: (g, 0, 0)),
            full((13, 2 * _D, _D)), full((13, 1, _D)),
            full((1, 2 * _D, 2)), full((1, 1, 2)),
        ],
        out_specs=pl.BlockSpec((1, _N, 2), lambda g: (g, 0, 0)),
    )
    return pl.pallas_call(
        _gcn_body,
        grid_spec=grid_spec,
        out_shape=jax.ShapeDtypeStruct((B, _N, 2), jnp.float32),
        compiler_params=pltpu.CompilerParams(
            dimension_semantics=("arbitrary",),
            vmem_limit_bytes=100 * 1024 * 1024,
        ),
    )(x, adj, w, b, v, c)
